# baseline (speedup 1.0000x reference)
import functools

import jax
import jax.numpy as jnp
from jax import lax
from jax.experimental import pallas as pl
from jax.experimental.pallas import tpu as pltpu
from jax.experimental.pallas import tpu_sc as plsc

N_SITES = 100000
K = 64
B = 16384
NC, NS, L = 2, 16, 16
NW = NC * NS
BPW = B // NW
CHUNK = 128
NCH = BPW // CHUNK

_D0 = 0.0009223163497825149
_D1 = 0.9797691943591391
_D2 = -0.3935581873890316
_D3 = 0.10669243657177084


def _softplus16(x):
    t = jnp.exp(-jnp.abs(x))
    p = _D2 + t * _D3
    p = _D1 + t * p
    p = _D0 + t * p
    return jnp.maximum(x, 0.0) + p


@functools.cache
def _get_mesh():
    return plsc.VectorSubcoreMesh(
        core_axis_name="c", subcore_axis_name="s", num_cores=NC, num_subcores=NS
    )


def _hbb_body(idx_hbm, qa_hbm, qb_hbm, out_hbm, idx_v, ih_v, ra_v, rb_v, gsems, wsem):
    wid = lax.axis_index("s") * NC + lax.axis_index("c")

    pltpu.sync_copy(idx_hbm.at[wid], idx_v)

    for j in range(NCH):
        for g in range(CHUNK // L):
            sl = pl.ds(g * L, L)
            ih_v[j, sl] = lax.shift_right_logical(idx_v[j, sl], 1)

    def fire(j):
        buf = j % 2
        return (
            pltpu.async_copy(qa_hbm.at[ih_v.at[j]], ra_v.at[buf], gsems.at[j]),
            pltpu.async_copy(qb_hbm.at[ih_v.at[j]], rb_v.at[buf], gsems.at[j]),
        )

    inflight = {0: fire(0), 1: fire(1)}

    writes = []
    for j in range(NCH):
        buf = j % 2
        for c in inflight.pop(j):
            c.wait()

        @plsc.parallel_loop(0, CHUNK // L)
        def _(g, j=j, buf=buf):
            pv = (idx_v[j, pl.ds(g * L, L)] & 1) * K
            for k in range(L):
                r = g * L + k
                off = pv[k]
                for c in range(K // L):
                    asl = pl.ds(off + c * L, L)
                    a = _softplus16(ra_v[buf, r, asl])
                    b = _softplus16(rb_v[buf, r, asl])
                    ra_v[buf, r, pl.ds(c * L, L)] = a / (a + b)

        wr = pltpu.async_copy(
            ra_v.at[buf],
            out_hbm.at[pl.ds(wid * BPW + j * CHUNK, CHUNK)],
            wsem,
        )
        if j + 2 < NCH:
            wr.wait()
            inflight[j + 2] = fire(j + 2)
        else:
            writes.append(wr)

    for wr in writes:
        wr.wait()


@functools.cache
def _get_hbb_sc():
    return functools.partial(
        pl.kernel,
        out_type=jax.ShapeDtypeStruct((B, 2 * K), jnp.float32),
        mesh=_get_mesh(),
        scratch_types=[
            pltpu.VMEM((NCH, CHUNK), jnp.int32),
            pltpu.VMEM((NCH, CHUNK), jnp.int32),
            pltpu.VMEM((2, CHUNK, 2 * K), jnp.float32),
            pltpu.VMEM((2, CHUNK, 2 * K), jnp.float32),
            pltpu.SemaphoreType.DMA((NCH,)),
            pltpu.SemaphoreType.DMA,
        ],
    )(_hbb_body)


def kernel(site_idx, q_a_site, q_b_site):
    qa2 = q_a_site.reshape(N_SITES // 2, 2 * K)
    qb2 = q_b_site.reshape(N_SITES // 2, 2 * K)
    idx = site_idx.astype(jnp.int32).reshape(NW, NCH, CHUNK)
    wide = _get_hbb_sc()(idx, qa2, qb2)
    return wide[:, :K]


# speedup vs baseline: 1.8302x; 1.8302x over previous
import functools

import jax
import jax.numpy as jnp
from jax import lax
from jax.experimental import pallas as pl
from jax.experimental.pallas import tpu as pltpu
from jax.experimental.pallas import tpu_sc as plsc

N_SITES = 100000
K = 64
B = 16384
NC, NS, L = 2, 16, 16
NW = NC * NS
BPW = B // NW
CHUNK = 128
NCH = BPW // CHUNK

_D0 = 0.0009223163497825149
_D1 = 0.9797691943591391
_D2 = -0.3935581873890316
_D3 = 0.10669243657177084


def _softplus16(x):
    t = jnp.exp(-jnp.abs(x))
    p = _D2 + t * _D3
    p = _D1 + t * p
    p = _D0 + t * p
    return jnp.maximum(x, 0.0) + p


@functools.cache
def _get_mesh():
    return plsc.VectorSubcoreMesh(
        core_axis_name="c", subcore_axis_name="s", num_cores=NC, num_subcores=NS
    )


def _hbb_body(idx_hbm, qab_hbm, out_hbm, idx_v, rows_v, gsems, wsem):
    wid = lax.axis_index("s") * NC + lax.axis_index("c")

    pltpu.sync_copy(idx_hbm.at[wid], idx_v)

    gathers = []
    for j in range(NCH):
        gathers.append(
            pltpu.async_copy(
                qab_hbm.at[idx_v.at[j]],
                rows_v.at[pl.ds(j * CHUNK, CHUNK)],
                gsems.at[j],
            )
        )

    writes = []
    for j in range(NCH):
        gathers[j].wait()

        @plsc.parallel_loop(j * CHUNK, (j + 1) * CHUNK, unroll=2)
        def _(r):
            for c in range(K // L):
                a = _softplus16(rows_v[r, pl.ds(c * L, L)])
                b = _softplus16(rows_v[r, pl.ds(K + c * L, L)])
                rows_v[r, pl.ds(c * L, L)] = a / (a + b)

        sl = pl.ds(j * CHUNK, CHUNK)
        writes.append(
            pltpu.async_copy(
                rows_v.at[sl], out_hbm.at[pl.ds(wid * BPW + j * CHUNK, CHUNK)], wsem
            )
        )

    for wr in writes:
        wr.wait()


@functools.cache
def _get_hbb_sc():
    return functools.partial(
        pl.kernel,
        out_type=jax.ShapeDtypeStruct((B, 2 * K), jnp.float32),
        mesh=_get_mesh(),
        scratch_types=[
            pltpu.VMEM((NCH, CHUNK), jnp.int32),
            pltpu.VMEM((BPW, 2 * K), jnp.float32),
            pltpu.SemaphoreType.DMA((NCH,)),
            pltpu.SemaphoreType.DMA,
        ],
    )(_hbb_body)


def kernel(site_idx, q_a_site, q_b_site):
    qab = jnp.concatenate([q_a_site, q_b_site], axis=1)
    idx = site_idx.astype(jnp.int32).reshape(NW, NCH, CHUNK)
    wide = _get_hbb_sc()(idx, qab)
    return wide[:, :K]
